# baseline (speedup 1.0000x reference)
import functools

import jax
import jax.numpy as jnp
from jax import lax
from jax.experimental import pallas as pl
from jax.experimental.pallas import tpu as pltpu
from jax.experimental.pallas import tpu_sc as plsc

NC, NS = 2, 16
LANE = 128
SUP = 8
SUPC = 64
G = 128
SHIFT = 14


def _sc_agg_scalar(x2d, src2d, dst2d, zeros2d):
    m = x2d.shape[0]
    rows = src2d.shape[0]
    rows_per_tile = rows // (NC * NS)
    nsup = rows_per_tile // SUP
    mesh = plsc.VectorSubcoreMesh(core_axis_name="c", subcore_axis_name="s")

    @functools.partial(
        pl.kernel,
        out_type=[jax.ShapeDtypeStruct((m, LANE), jnp.float32),
                  jax.ShapeDtypeStruct((m, LANE), jnp.float32)],
        mesh=mesh,
        scratch_types=[
            pltpu.VMEM((m, LANE), jnp.float32),
            pltpu.VMEM((m, LANE), jnp.float32),
            pltpu.VMEM((SUP, LANE), jnp.int32),
            pltpu.VMEM((SUP, LANE), jnp.int32),
            pltpu.VMEM((m,), jnp.int32),
            pltpu.VMEM_SHARED((m, LANE), jnp.float32),
            pltpu.SemaphoreType.DMA,
        ],
        compiler_params=pltpu.CompilerParams(needs_layout_passes=False),
    )
    def k(xr, srcr, dstr, zr, o0, o1, xloc, acc, src_sb, dst_sb, ridx, sacc,
          sem):
        c = lax.axis_index("c")
        s = lax.axis_index("s")
        tid = c * NS + s

        pltpu.sync_copy(xr, xloc)

        def zero_row(r, carry):
            for l in range(LANE // 16):
                acc[r, pl.ds(l * 16, 16)] = jnp.zeros((16,), jnp.float32)
            return carry
        lax.fori_loop(0, m, zero_row, 0)
        for kk in range(m // 16):
            ridx[pl.ds(kk * 16, 16)] = lax.iota(jnp.int32, 16) + kk * 16

        @pl.when(s == 0)
        def _():
            pltpu.sync_copy(zr, sacc)
        plsc.subcore_barrier()

        def body(sup, carry):
            r0 = tid * rows_per_tile + sup * SUP
            pltpu.sync_copy(srcr.at[pl.ds(r0, SUP)], src_sb)
            pltpu.sync_copy(dstr.at[pl.ds(r0, SUP)], dst_sb)
            for j in range(SUP):
                for l in range(LANE // 16):
                    sv = src_sb[j, pl.ds(l * 16, 16)]
                    dv = dst_sb[j, pl.ds(l * 16, 16)]
                    vals = plsc.load_gather(
                        xloc, [lax.shift_right_logical(sv, 7), sv & 127])
                    plsc.addupdate_scatter(
                        acc, [lax.shift_right_logical(dv, 7), dv & 127], vals)
            return carry
        lax.fori_loop(0, nsup, body, 0)

        pltpu.sync_copy(acc, sacc.at[ridx], add=True)
        plsc.subcore_barrier()

        @pl.when((c == 0) & (s == 0))
        def _():
            pltpu.sync_copy(sacc, o0)

        @pl.when((c == 1) & (s == 0))
        def _():
            pltpu.sync_copy(sacc, o1)

    return k(x2d, src2d, dst2d, zeros2d)


def _sc_agg_cols(ht3, pk2d):
    nt = ht3.shape[0]
    cb = ht3.shape[1]
    mm = ht3.shape[2]
    rows = pk2d.shape[0]
    rows_per_tile = rows
    nsup = rows // SUPC
    mesh = plsc.VectorSubcoreMesh(core_axis_name="c", subcore_axis_name="s")
    dmask = (1 << SHIFT) - 1

    @functools.partial(
        pl.kernel,
        out_type=jax.ShapeDtypeStruct((nt, cb, mm), jnp.float32),
        mesh=mesh,
        scratch_types=[
            pltpu.VMEM((cb, mm), jnp.float32),
            pltpu.VMEM((cb, mm), jnp.float32),
            pltpu.VMEM((2, SUPC, LANE), jnp.int32),
            pltpu.SemaphoreType.DMA,
            pltpu.SemaphoreType.DMA,
        ],
        compiler_params=pltpu.CompilerParams(needs_layout_passes=False),
    )
    def k(htr, pkr, out, hloc, acc, pk_sb, sem0, sem1):
        c = lax.axis_index("c")
        s = lax.axis_index("s")
        tid = c * NS + s

        pltpu.sync_copy(htr.at[tid], hloc)

        def zero_col(i, carry):
            for r in range(cb):
                acc[r, pl.ds(i * 16, 16)] = jnp.zeros((16,), jnp.float32)
            return carry
        lax.fori_loop(0, mm // 16, zero_col, 0)

        rowidx = [jnp.full((16,), r, jnp.int32) for r in range(cb)]

        def process(buf):
            def prow(j, carry):
                for l in range(LANE // 16):
                    pv = pk_sb[buf, j, pl.ds(l * 16, 16)]
                    sv = lax.shift_right_logical(pv, SHIFT)
                    dv = pv & dmask
                    for r in range(cb):
                        vals = plsc.load_gather(hloc, [rowidx[r], sv])
                        plsc.addupdate_scatter(acc, [rowidx[r], dv], vals)
                return carry
            lax.fori_loop(0, SUPC, prow, 0)

        pltpu.async_copy(pkr.at[pl.ds(0, SUPC)], pk_sb.at[0], sem0)

        def body(i, carry):
            g0 = i * 2

            @pl.when(g0 + 1 < nsup)
            def _():
                pltpu.async_copy(pkr.at[pl.ds((g0 + 1) * SUPC, SUPC)],
                                 pk_sb.at[1], sem1)
            pltpu.make_async_copy(pkr.at[pl.ds(g0 * SUPC, SUPC)],
                                  pk_sb.at[0], sem0).wait()
            process(0)

            @pl.when(g0 + 2 < nsup)
            def _():
                pltpu.async_copy(pkr.at[pl.ds((g0 + 2) * SUPC, SUPC)],
                                 pk_sb.at[0], sem0)

            @pl.when(g0 + 1 < nsup)
            def _():
                pltpu.make_async_copy(pkr.at[pl.ds((g0 + 1) * SUPC, SUPC)],
                                      pk_sb.at[1], sem1).wait()
                process(1)
            return carry

        lax.fori_loop(0, (nsup + 1) // 2, body, 0)

        pltpu.sync_copy(acc, out.at[tid])

    return k(ht3, pk2d)


def _tc_layer1(a1, x, w_rel, w_root, b):
    n = x.shape[0]
    h = w_rel.shape[1]
    bn = 1000

    def body(a1r, xr, wr, wt, br, out):
        hv = a1r[...] * wr[...] + xr[...] * wt[...] + br[...]
        out[...] = jnp.maximum(hv, 0.0)

    return pl.pallas_call(
        body,
        grid=(n // bn,),
        in_specs=[
            pl.BlockSpec((bn, 1), lambda i: (i, 0)),
            pl.BlockSpec((bn, 1), lambda i: (i, 0)),
            pl.BlockSpec((1, h), lambda i: (0, 0)),
            pl.BlockSpec((1, h), lambda i: (0, 0)),
            pl.BlockSpec((1, h), lambda i: (0, 0)),
        ],
        out_specs=pl.BlockSpec((bn, h), lambda i: (i, 0)),
        out_shape=jax.ShapeDtypeStruct((n, h), jnp.float32),
    )(a1, x, w_rel, w_root, b)


def _tc_dense(g, hin, w_rel, w_root, b, relu):
    n = g.shape[0]
    h = w_rel.shape[0]
    bn = 1000

    def body(gr, hr, wrr, wtr, br, out):
        hv = (jnp.dot(gr[...], wrr[...], preferred_element_type=jnp.float32)
              + jnp.dot(hr[...], wtr[...], preferred_element_type=jnp.float32)
              + br[...])
        if relu:
            hv = jnp.maximum(hv, 0.0)
        out[...] = hv

    return pl.pallas_call(
        body,
        grid=(n // bn,),
        in_specs=[
            pl.BlockSpec((bn, h), lambda i: (i, 0)),
            pl.BlockSpec((bn, h), lambda i: (i, 0)),
            pl.BlockSpec((h, h), lambda i: (0, 0)),
            pl.BlockSpec((h, h), lambda i: (0, 0)),
            pl.BlockSpec((1, h), lambda i: (0, 0)),
        ],
        out_specs=pl.BlockSpec((bn, h), lambda i: (i, 0)),
        out_shape=jax.ShapeDtypeStruct((n, h), jnp.float32),
    )(g, hin, w_rel, w_root, b)


def _tc_final(g, hin, w_rel, w_root, b, batch3d, w_lin, b_lin):
    n = g.shape[0]
    h = w_rel.shape[0]
    o = w_lin.shape[1]
    bn = 1000
    nb = n // bn

    def body(gr, hr, wrr, wtr, br, batr, wlr, blr, out, psum, cnt):
        i = pl.program_id(0)
        hv = (jnp.dot(gr[...], wrr[...], preferred_element_type=jnp.float32)
              + jnp.dot(hr[...], wtr[...], preferred_element_type=jnp.float32)
              + br[...])
        ids = batr[0]
        gi = lax.broadcasted_iota(jnp.int32, (G, bn), 0)
        onehot = jnp.where(jnp.broadcast_to(ids, (G, bn)) == gi,
                           jnp.float32(1.0), jnp.float32(0.0))
        ps = jnp.dot(onehot, hv, preferred_element_type=jnp.float32)
        ct = jnp.dot(onehot, jnp.ones((bn, h), jnp.float32),
                     preferred_element_type=jnp.float32)

        @pl.when(i == 0)
        def _():
            psum[...] = ps
            cnt[...] = ct

        @pl.when(i > 0)
        def _():
            psum[...] += ps
            cnt[...] += ct

        @pl.when(i == nb - 1)
        def _():
            pooled = psum[...] / jnp.maximum(cnt[...], 1.0)
            out[...] = jnp.dot(pooled, wlr[...],
                               preferred_element_type=jnp.float32) + blr[...]

    return pl.pallas_call(
        body,
        grid=(nb,),
        in_specs=[
            pl.BlockSpec((bn, h), lambda i: (i, 0)),
            pl.BlockSpec((bn, h), lambda i: (i, 0)),
            pl.BlockSpec((h, h), lambda i: (0, 0)),
            pl.BlockSpec((h, h), lambda i: (0, 0)),
            pl.BlockSpec((1, h), lambda i: (0, 0)),
            pl.BlockSpec((1, 1, bn), lambda i: (i, 0, 0)),
            pl.BlockSpec((h, o), lambda i: (0, 0)),
            pl.BlockSpec((1, o), lambda i: (0, 0)),
        ],
        out_specs=pl.BlockSpec((G, o), lambda i: (0, 0)),
        out_shape=jax.ShapeDtypeStruct((G, o), jnp.float32),
        scratch_shapes=[
            pltpu.VMEM((G, h), jnp.float32),
            pltpu.VMEM((G, h), jnp.float32),
        ],
    )(g, hin, w_rel, w_root, b, batch3d, w_lin, b_lin)


def kernel(x, edge_index, batch, W_rel1, b_rel1, W_root1, W_rel2, b_rel2,
           W_root2, W_rel3, b_rel3, W_root3, W_lin, b_lin):
    n = x.shape[0]
    e = edge_index.shape[1]
    h = W_rel2.shape[0]
    o = W_lin.shape[1]
    nt = NC * NS
    cb = h // nt

    tile = nt * SUP * LANE
    ep = ((e + tile - 1) // tile) * tile
    src = edge_index[0]
    dst = edge_index[1]
    srcp = jnp.concatenate(
        [src, jnp.zeros((ep - e,), jnp.int32)]).reshape(ep // LANE, LANE)
    dstp = jnp.concatenate(
        [dst, jnp.full((ep - e,), n, jnp.int32)]).reshape(ep // LANE, LANE)
    pk2d = (srcp << SHIFT) | dstp

    m = -((n + 1) // -LANE)
    m = -(m // -16) * 16
    x2d = jnp.concatenate(
        [x[:, 0], jnp.zeros((m * LANE - n,), jnp.float32)]).reshape(m, LANE)
    zeros2d = jnp.zeros((m, LANE), jnp.float32)

    mm = m * LANE

    def to_blocked(hmat):
        htp = jnp.pad(hmat.T, ((0, 0), (0, mm - n)))
        return htp.reshape(nt, cb, mm)

    def from_blocked(hb):
        return hb.reshape(h, mm)[:, :n].T

    b1 = b_rel1.reshape(1, h)
    b2 = b_rel2.reshape(1, h)
    b3 = b_rel3.reshape(1, h)
    bl = b_lin.reshape(1, o)
    batch3d = batch.reshape(n // 1000, 1, 1000)

    p0, p1 = _sc_agg_scalar(x2d, srcp, dstp, zeros2d)
    a1 = (p0 + p1).reshape(m * LANE)[:n].reshape(n, 1)
    h1 = _tc_layer1(a1, x, W_rel1, W_root1, b1)
    g2 = from_blocked(_sc_agg_cols(to_blocked(h1), pk2d))
    h2 = _tc_dense(g2, h1, W_rel2, W_root2, b2, relu=True)
    g3 = from_blocked(_sc_agg_cols(to_blocked(h2), pk2d))
    return _tc_final(g3, h2, W_rel3, W_root3, b3, batch3d, W_lin, bl)


# speedup vs baseline: 2.1798x; 2.1798x over previous
import functools

import jax
import jax.numpy as jnp
from jax import lax
from jax.experimental import pallas as pl
from jax.experimental.pallas import tpu as pltpu
from jax.experimental.pallas import tpu_sc as plsc

NC, NS = 2, 16
LANE = 128
SUP = 8
G = 128


def _sc_agg_scalar(x2d, src2d, dst2d, zeros2d):
    m = x2d.shape[0]
    rows = src2d.shape[0]
    rows_per_tile = rows // (NC * NS)
    nsup = rows_per_tile // SUP
    mesh = plsc.VectorSubcoreMesh(core_axis_name="c", subcore_axis_name="s")

    @functools.partial(
        pl.kernel,
        out_type=[jax.ShapeDtypeStruct((m, LANE), jnp.float32),
                  jax.ShapeDtypeStruct((m, LANE), jnp.float32)],
        mesh=mesh,
        scratch_types=[
            pltpu.VMEM((m, LANE), jnp.float32),
            pltpu.VMEM((m, LANE), jnp.float32),
            pltpu.VMEM((SUP, LANE), jnp.int32),
            pltpu.VMEM((SUP, LANE), jnp.int32),
            pltpu.VMEM((m,), jnp.int32),
            pltpu.VMEM_SHARED((m, LANE), jnp.float32),
            pltpu.SemaphoreType.DMA,
        ],
        compiler_params=pltpu.CompilerParams(needs_layout_passes=False),
    )
    def k(xr, srcr, dstr, zr, o0, o1, xloc, acc, src_sb, dst_sb, ridx, sacc,
          sem):
        c = lax.axis_index("c")
        s = lax.axis_index("s")
        tid = c * NS + s

        pltpu.sync_copy(xr, xloc)

        def zero_row(r, carry):
            for l in range(LANE // 16):
                acc[r, pl.ds(l * 16, 16)] = jnp.zeros((16,), jnp.float32)
            return carry
        lax.fori_loop(0, m, zero_row, 0)
        for kk in range(m // 16):
            ridx[pl.ds(kk * 16, 16)] = lax.iota(jnp.int32, 16) + kk * 16

        @pl.when(s == 0)
        def _():
            pltpu.sync_copy(zr, sacc)
        plsc.subcore_barrier()

        def body(sup, carry):
            r0 = tid * rows_per_tile + sup * SUP
            pltpu.sync_copy(srcr.at[pl.ds(r0, SUP)], src_sb)
            pltpu.sync_copy(dstr.at[pl.ds(r0, SUP)], dst_sb)
            for j in range(SUP):
                for l in range(LANE // 16):
                    sv = src_sb[j, pl.ds(l * 16, 16)]
                    dv = dst_sb[j, pl.ds(l * 16, 16)]
                    vals = plsc.load_gather(
                        xloc, [lax.shift_right_logical(sv, 7), sv & 127])
                    plsc.addupdate_scatter(
                        acc, [lax.shift_right_logical(dv, 7), dv & 127], vals)
            return carry
        lax.fori_loop(0, nsup, body, 0)

        pltpu.sync_copy(acc, sacc.at[ridx], add=True)
        plsc.subcore_barrier()

        @pl.when((c == 0) & (s == 0))
        def _():
            pltpu.sync_copy(sacc, o0)

        @pl.when((c == 1) & (s == 0))
        def _():
            pltpu.sync_copy(sacc, o1)

    return k(x2d, src2d, dst2d, zeros2d)


SHIFT = 14


def _sc_route(pk2d, n):
    nh = n // 2
    rows = pk2d.shape[0]
    rpt = rows // (NC * NS)
    caprows = rpt + 8
    dmask = (1 << SHIFT) - 1
    adj = [0, nh, nh << SHIFT, (nh << SHIFT) + nh]
    dummy_base = (nh << SHIFT) + nh
    mesh = plsc.VectorSubcoreMesh(core_axis_name="c", subcore_axis_name="s")

    @functools.partial(
        pl.kernel,
        out_type=(
            [jax.ShapeDtypeStruct((NC * NS, caprows, LANE), jnp.int32)] * 4
            + [jax.ShapeDtypeStruct((NC * NS, 16), jnp.int32)] * 4),
        mesh=mesh,
        scratch_types=[
            pltpu.VMEM((rpt, LANE), jnp.int32),
            pltpu.VMEM((caprows, LANE), jnp.int32),
            pltpu.VMEM((caprows, LANE), jnp.int32),
            pltpu.VMEM((caprows, LANE), jnp.int32),
            pltpu.VMEM((caprows, LANE), jnp.int32),
            pltpu.VMEM((16,), jnp.int32),
            pltpu.SemaphoreType.DMA,
        ],
        compiler_params=pltpu.CompilerParams(needs_layout_passes=False),
    )
    def k(pkr, l0, l1, l2, l3, c0, c1, c2, c3, pkb, b0, b1, b2, b3, cb, sem):
        c = lax.axis_index("c")
        s = lax.axis_index("s")
        tid = c * NS + s
        lbufs = (b0, b1, b2, b3)
        louts = (l0, l1, l2, l3)
        couts = (c0, c1, c2, c3)

        pltpu.sync_copy(pkr.at[pl.ds(tid * rpt, rpt)], pkb)

        def row(j, offs):
            for l in range(LANE // 16):
                pv = pkb[j, pl.ds(l * 16, 16)]
                sv = lax.shift_right_logical(pv, SHIFT)
                dv = pv & dmask
                ms = sv < nh
                md = dv < nh
                new = []
                for q in range(4):
                    mq = (ms if q < 2 else jnp.logical_not(ms)) & (
                        md if q % 2 == 0 else jnp.logical_not(md))
                    mi = mq.astype(jnp.int32)
                    pos = offs[q] + plsc.cumsum(mi) - 1
                    plsc.store_scatter(
                        lbufs[q],
                        [lax.shift_right_logical(pos, 7), pos & 127],
                        pv - adj[q], mask=mq)
                    new.append(offs[q] + jnp.sum(mi))
                offs = tuple(new)
            return offs

        offs = lax.fori_loop(
            0, rpt, row,
            (jnp.int32(0), jnp.int32(0), jnp.int32(0), jnp.int32(0)))

        dummy16 = dummy_base + lax.iota(jnp.int32, 16) * 8
        for q in range(4):
            off = offs[q]
            for kk in range(LANE * 8 // 16):
                pos = off + kk * 16 + lax.iota(jnp.int32, 16)
                plsc.store_scatter(
                    lbufs[q],
                    [lax.shift_right_logical(pos, 7), pos & 127], dummy16)
            ra = (off + LANE * 8 - 1) // (LANE * 8) * 8
            cb[pl.ds(0, 16)] = jnp.zeros((16,), jnp.int32) + ra
            pltpu.sync_copy(cb, couts[q].at[tid])

            def wout(cw, carry):
                pltpu.sync_copy(lbufs[q].at[pl.ds(cw * 8, 8)],
                                louts[q].at[tid, pl.ds(cw * 8, 8)])
                return carry
            lax.fori_loop(0, ra // 8, wout, 0)

    return k(pk2d)


def _sc_agg_quad(h, lsts, cnts, zeros):
    n = h.shape[0]
    w = h.shape[1]
    nh = n // 2
    za = nh + LANE
    cs = (nh // (NS * 8)) * 8
    rems = nh - NS * cs
    ca = (za // (NS * 8)) * 8
    dmask = (1 << SHIFT) - 1
    caprows = lsts[0].shape[1]
    mesh = plsc.VectorSubcoreMesh(core_axis_name="c", subcore_axis_name="s")

    @functools.partial(
        pl.kernel,
        out_type=[jax.ShapeDtypeStruct((n, w), jnp.float32),
                  jax.ShapeDtypeStruct((n, w), jnp.float32)],
        mesh=mesh,
        scratch_types=[
            pltpu.VMEM((SUP, LANE), jnp.int32),
            pltpu.VMEM((SUP, LANE), jnp.int32),
            pltpu.VMEM((SUP, LANE), jnp.int32),
            pltpu.VMEM((2, LANE, w), jnp.float32),
            pltpu.VMEM((16,), jnp.int32),
            pltpu.VMEM_SHARED((za, w), jnp.float32),
            pltpu.VMEM_SHARED((nh + 8, w), jnp.float32),
            pltpu.SemaphoreType.DMA,
            pltpu.SemaphoreType.DMA,
        ],
        compiler_params=pltpu.CompilerParams(needs_layout_passes=False),
    )
    def k(hr, l0, l1, l2, l3, c0r, c1r, c2r, c3r, zr, o0, o1,
          pk_sb, lsrc_sb, ldst_sb, rbuf, cntv, acc, stage, sem0, sem1):
        c = lax.axis_index("c")
        s = lax.axis_index("s")
        base = c * nh
        sems = (sem0, sem1)

        pltpu.sync_copy(hr.at[pl.ds(base + s * cs, cs)],
                        stage.at[pl.ds(s * cs, cs)])

        @pl.when(s == 0)
        def _():
            pltpu.sync_copy(hr.at[pl.ds(base + NS * cs, rems)],
                            stage.at[pl.ds(NS * cs, rems)])
            pltpu.sync_copy(zr.at[pl.ds(0, 8)], stage.at[pl.ds(nh, 8)])

        def do_regions(lref, cref):
            for r in range(2):
                t = s * 2 + r
                pltpu.sync_copy(cref.at[t], cntv)
                nsup_t = jnp.max(cntv[...]) // 8

                def body(sup, carry):
                    pltpu.sync_copy(lref.at[t, pl.ds(sup * SUP, SUP)], pk_sb)
                    for j in range(SUP):
                        for l in range(LANE // 16):
                            pv = pk_sb[j, pl.ds(l * 16, 16)]
                            lsrc_sb[j, pl.ds(l * 16, 16)] = (
                                lax.shift_right_logical(pv, SHIFT))
                            ldst_sb[j, pl.ds(l * 16, 16)] = pv & dmask
                    descs = [pltpu.async_copy(stage.at[lsrc_sb.at[0]],
                                              rbuf.at[0], sems[0])]
                    for j in range(SUP):
                        if j + 1 < SUP:
                            descs.append(
                                pltpu.async_copy(stage.at[lsrc_sb.at[j + 1]],
                                                 rbuf.at[(j + 1) % 2],
                                                 sems[(j + 1) % 2]))
                        descs[j].wait()
                        pltpu.sync_copy(rbuf.at[j % 2],
                                        acc.at[ldst_sb.at[j]], add=True)
                    return carry
                lax.fori_loop(0, nsup_t, body, 0)

        for p in range(2):
            pltpu.sync_copy(zr.at[pl.ds(s * ca, ca)],
                            acc.at[pl.ds(s * ca, ca)])

            @pl.when(s == 0)
            def _():
                pltpu.sync_copy(zr.at[pl.ds(NS * ca, za - NS * ca)],
                                acc.at[pl.ds(NS * ca, za - NS * ca)])
            plsc.subcore_barrier()

            @pl.when(c == 0)
            def _():
                do_regions((l0, l1)[p], (c0r, c1r)[p])

            @pl.when(c == 1)
            def _():
                do_regions((l3, l2)[p], (c3r, c2r)[p])
            plsc.subcore_barrier()

            dbase = (c if p == 0 else 1 - c) * nh

            @pl.when(c == 0)
            def _():
                pltpu.sync_copy(acc.at[pl.ds(s * cs, cs)],
                                o0.at[pl.ds(dbase + s * cs, cs)])

                @pl.when(s == 0)
                def _():
                    pltpu.sync_copy(acc.at[pl.ds(NS * cs, rems)],
                                    o0.at[pl.ds(dbase + NS * cs, rems)])

            @pl.when(c == 1)
            def _():
                pltpu.sync_copy(acc.at[pl.ds(s * cs, cs)],
                                o1.at[pl.ds(dbase + s * cs, cs)])

                @pl.when(s == 0)
                def _():
                    pltpu.sync_copy(acc.at[pl.ds(NS * cs, rems)],
                                    o1.at[pl.ds(dbase + NS * cs, rems)])
            plsc.subcore_barrier()

    return k(h, *lsts, *cnts, zeros)


def _tc_layer1(a1, x, w_rel, w_root, b):
    n = x.shape[0]
    h = w_rel.shape[1]
    bn = 1000

    def body(a1r, xr, wr, wt, br, out):
        hv = a1r[...] * wr[...] + xr[...] * wt[...] + br[...]
        out[...] = jnp.maximum(hv, 0.0)

    return pl.pallas_call(
        body,
        grid=(n // bn,),
        in_specs=[
            pl.BlockSpec((bn, 1), lambda i: (i, 0)),
            pl.BlockSpec((bn, 1), lambda i: (i, 0)),
            pl.BlockSpec((1, h), lambda i: (0, 0)),
            pl.BlockSpec((1, h), lambda i: (0, 0)),
            pl.BlockSpec((1, h), lambda i: (0, 0)),
        ],
        out_specs=pl.BlockSpec((bn, h), lambda i: (i, 0)),
        out_shape=jax.ShapeDtypeStruct((n, h), jnp.float32),
    )(a1, x, w_rel, w_root, b)


def _tc_dense(g0, g1, hin, w_rel, w_root, b, relu):
    n = g0.shape[0]
    h = w_rel.shape[0]
    bn = 1000

    def body(g0r, g1r, hr, wrr, wtr, br, out):
        g = g0r[...] + g1r[...]
        hv = (jnp.dot(g, wrr[...], preferred_element_type=jnp.float32)
              + jnp.dot(hr[...], wtr[...], preferred_element_type=jnp.float32)
              + br[...])
        if relu:
            hv = jnp.maximum(hv, 0.0)
        out[...] = hv

    return pl.pallas_call(
        body,
        grid=(n // bn,),
        in_specs=[
            pl.BlockSpec((bn, h), lambda i: (i, 0)),
            pl.BlockSpec((bn, h), lambda i: (i, 0)),
            pl.BlockSpec((bn, h), lambda i: (i, 0)),
            pl.BlockSpec((h, h), lambda i: (0, 0)),
            pl.BlockSpec((h, h), lambda i: (0, 0)),
            pl.BlockSpec((1, h), lambda i: (0, 0)),
        ],
        out_specs=pl.BlockSpec((bn, h), lambda i: (i, 0)),
        out_shape=jax.ShapeDtypeStruct((n, h), jnp.float32),
    )(g0, g1, hin, w_rel, w_root, b)


def _tc_final(g0, g1, hin, w_rel, w_root, b, batch3d, w_lin, b_lin):
    n = g0.shape[0]
    h = w_rel.shape[0]
    o = w_lin.shape[1]
    bn = 1000
    nb = n // bn

    def body(g0r, g1r, hr, wrr, wtr, br, batr, wlr, blr, out, psum, cnt):
        i = pl.program_id(0)
        g = g0r[...] + g1r[...]
        hv = (jnp.dot(g, wrr[...], preferred_element_type=jnp.float32)
              + jnp.dot(hr[...], wtr[...], preferred_element_type=jnp.float32)
              + br[...])
        ids = batr[0]
        gi = lax.broadcasted_iota(jnp.int32, (G, bn), 0)
        onehot = jnp.where(jnp.broadcast_to(ids, (G, bn)) == gi,
                           jnp.float32(1.0), jnp.float32(0.0))
        ps = jnp.dot(onehot, hv, preferred_element_type=jnp.float32)
        ct = jnp.dot(onehot, jnp.ones((bn, h), jnp.float32),
                     preferred_element_type=jnp.float32)

        @pl.when(i == 0)
        def _():
            psum[...] = ps
            cnt[...] = ct

        @pl.when(i > 0)
        def _():
            psum[...] += ps
            cnt[...] += ct

        @pl.when(i == nb - 1)
        def _():
            pooled = psum[...] / jnp.maximum(cnt[...], 1.0)
            out[...] = jnp.dot(pooled, wlr[...],
                               preferred_element_type=jnp.float32) + blr[...]

    return pl.pallas_call(
        body,
        grid=(nb,),
        in_specs=[
            pl.BlockSpec((bn, h), lambda i: (i, 0)),
            pl.BlockSpec((bn, h), lambda i: (i, 0)),
            pl.BlockSpec((bn, h), lambda i: (i, 0)),
            pl.BlockSpec((h, h), lambda i: (0, 0)),
            pl.BlockSpec((h, h), lambda i: (0, 0)),
            pl.BlockSpec((1, h), lambda i: (0, 0)),
            pl.BlockSpec((1, 1, bn), lambda i: (i, 0, 0)),
            pl.BlockSpec((h, o), lambda i: (0, 0)),
            pl.BlockSpec((1, o), lambda i: (0, 0)),
        ],
        out_specs=pl.BlockSpec((G, o), lambda i: (0, 0)),
        out_shape=jax.ShapeDtypeStruct((G, o), jnp.float32),
        scratch_shapes=[
            pltpu.VMEM((G, h), jnp.float32),
            pltpu.VMEM((G, h), jnp.float32),
        ],
    )(g0, g1, hin, w_rel, w_root, b, batch3d, w_lin, b_lin)


def kernel(x, edge_index, batch, W_rel1, b_rel1, W_root1, W_rel2, b_rel2,
           W_root2, W_rel3, b_rel3, W_root3, W_lin, b_lin):
    n = x.shape[0]
    e = edge_index.shape[1]
    h = W_rel2.shape[0]
    o = W_lin.shape[1]

    tile = NC * NS * SUP * LANE
    ep = ((e + tile - 1) // tile) * tile
    src = edge_index[0]
    dst = edge_index[1]
    srcp = jnp.concatenate(
        [src, jnp.zeros((ep - e,), jnp.int32)]).reshape(ep // LANE, LANE)
    dstp = jnp.concatenate(
        [dst, jnp.full((ep - e,), n, jnp.int32)]).reshape(ep // LANE, LANE)
    pk2d = (srcp << SHIFT) | dstp
    zeros = jnp.zeros((n + 8, h), jnp.float32)

    m = -((n + 1) // -LANE)
    m = -(m // -16) * 16
    x2d = jnp.concatenate(
        [x[:, 0], jnp.zeros((m * LANE - n,), jnp.float32)]).reshape(m, LANE)
    zeros2d = jnp.zeros((m, LANE), jnp.float32)

    b1 = b_rel1.reshape(1, h)
    b2 = b_rel2.reshape(1, h)
    b3 = b_rel3.reshape(1, h)
    bl = b_lin.reshape(1, o)
    batch3d = batch.reshape(n // 1000, 1, 1000)

    route = _sc_route(pk2d, n)
    lsts, cnts = tuple(route[:4]), tuple(route[4:])
    p0, p1 = _sc_agg_scalar(x2d, srcp, dstp, zeros2d)
    a1 = (p0 + p1).reshape(m * LANE)[:n].reshape(n, 1)
    h1 = _tc_layer1(a1, x, W_rel1, W_root1, b1)
    g20, g21 = _sc_agg_quad(h1, lsts, cnts, zeros)
    h2 = _tc_dense(g20, g21, h1, W_rel2, W_root2, b2, relu=True)
    g30, g31 = _sc_agg_quad(h2, lsts, cnts, zeros)
    return _tc_final(g30, g31, h2, W_rel3, W_root3, b3, batch3d, W_lin, bl)


# speedup vs baseline: 2.2309x; 1.0234x over previous
import functools

import jax
import jax.numpy as jnp
from jax import lax
from jax.experimental import pallas as pl
from jax.experimental.pallas import tpu as pltpu
from jax.experimental.pallas import tpu_sc as plsc

NC, NS = 2, 16
LANE = 128
SUP = 8
G = 128


def _sc_agg_scalar(x2d, src2d, dst2d, zeros2d):
    m = x2d.shape[0]
    rows = src2d.shape[0]
    rows_per_tile = rows // (NC * NS)
    nsup = rows_per_tile // SUP
    mesh = plsc.VectorSubcoreMesh(core_axis_name="c", subcore_axis_name="s")

    @functools.partial(
        pl.kernel,
        out_type=[jax.ShapeDtypeStruct((m, LANE), jnp.float32),
                  jax.ShapeDtypeStruct((m, LANE), jnp.float32)],
        mesh=mesh,
        scratch_types=[
            pltpu.VMEM((m, LANE), jnp.float32),
            pltpu.VMEM((m, LANE), jnp.float32),
            pltpu.VMEM((SUP, LANE), jnp.int32),
            pltpu.VMEM((SUP, LANE), jnp.int32),
            pltpu.VMEM((m,), jnp.int32),
            pltpu.VMEM_SHARED((m, LANE), jnp.float32),
            pltpu.SemaphoreType.DMA,
        ],
        compiler_params=pltpu.CompilerParams(needs_layout_passes=False),
    )
    def k(xr, srcr, dstr, zr, o0, o1, xloc, acc, src_sb, dst_sb, ridx, sacc,
          sem):
        c = lax.axis_index("c")
        s = lax.axis_index("s")
        tid = c * NS + s

        pltpu.sync_copy(xr, xloc)

        def zero_row(r, carry):
            for l in range(LANE // 16):
                acc[r, pl.ds(l * 16, 16)] = jnp.zeros((16,), jnp.float32)
            return carry
        lax.fori_loop(0, m, zero_row, 0)
        for kk in range(m // 16):
            ridx[pl.ds(kk * 16, 16)] = lax.iota(jnp.int32, 16) + kk * 16

        @pl.when(s == 0)
        def _():
            pltpu.sync_copy(zr, sacc)
        plsc.subcore_barrier()

        def body(sup, carry):
            r0 = tid * rows_per_tile + sup * SUP
            pltpu.sync_copy(srcr.at[pl.ds(r0, SUP)], src_sb)
            pltpu.sync_copy(dstr.at[pl.ds(r0, SUP)], dst_sb)
            for j in range(SUP):
                for l in range(LANE // 16):
                    sv = src_sb[j, pl.ds(l * 16, 16)]
                    dv = dst_sb[j, pl.ds(l * 16, 16)]
                    vals = plsc.load_gather(
                        xloc, [lax.shift_right_logical(sv, 7), sv & 127])
                    plsc.addupdate_scatter(
                        acc, [lax.shift_right_logical(dv, 7), dv & 127], vals)
            return carry
        lax.fori_loop(0, nsup, body, 0)

        pltpu.sync_copy(acc, sacc.at[ridx], add=True)
        plsc.subcore_barrier()

        @pl.when((c == 0) & (s == 0))
        def _():
            pltpu.sync_copy(sacc, o0)

        @pl.when((c == 1) & (s == 0))
        def _():
            pltpu.sync_copy(sacc, o1)

    return k(x2d, src2d, dst2d, zeros2d)


SHIFT = 14


def _sc_route(pk2d, n):
    nh = n // 2
    rows = pk2d.shape[0]
    rpt = rows // (NC * NS)
    caprows = rpt + 8
    dmask = (1 << SHIFT) - 1
    adj = [0, nh, nh << SHIFT, (nh << SHIFT) + nh]
    dummy_base = (nh << SHIFT) + nh
    mesh = plsc.VectorSubcoreMesh(core_axis_name="c", subcore_axis_name="s")

    @functools.partial(
        pl.kernel,
        out_type=(
            [jax.ShapeDtypeStruct((NC * NS, caprows, LANE), jnp.int32)] * 4
            + [jax.ShapeDtypeStruct((NC * NS, 16), jnp.int32)] * 4),
        mesh=mesh,
        scratch_types=[
            pltpu.VMEM((rpt, LANE), jnp.int32),
            pltpu.VMEM((caprows, LANE), jnp.int32),
            pltpu.VMEM((caprows, LANE), jnp.int32),
            pltpu.VMEM((caprows, LANE), jnp.int32),
            pltpu.VMEM((caprows, LANE), jnp.int32),
            pltpu.VMEM((16,), jnp.int32),
            pltpu.SemaphoreType.DMA,
        ],
        compiler_params=pltpu.CompilerParams(needs_layout_passes=False),
    )
    def k(pkr, l0, l1, l2, l3, c0, c1, c2, c3, pkb, b0, b1, b2, b3, cb, sem):
        c = lax.axis_index("c")
        s = lax.axis_index("s")
        tid = c * NS + s
        lbufs = (b0, b1, b2, b3)
        louts = (l0, l1, l2, l3)
        couts = (c0, c1, c2, c3)

        pltpu.sync_copy(pkr.at[pl.ds(tid * rpt, rpt)], pkb)

        def row(j, offs):
            for l in range(LANE // 16):
                pv = pkb[j, pl.ds(l * 16, 16)]
                sv = lax.shift_right_logical(pv, SHIFT)
                dv = pv & dmask
                ms = sv < nh
                md = dv < nh
                new = []
                for q in range(4):
                    mq = (ms if q < 2 else jnp.logical_not(ms)) & (
                        md if q % 2 == 0 else jnp.logical_not(md))
                    mi = mq.astype(jnp.int32)
                    pos = offs[q] + plsc.cumsum(mi) - 1
                    plsc.store_scatter(
                        lbufs[q],
                        [lax.shift_right_logical(pos, 7), pos & 127],
                        pv - adj[q], mask=mq)
                    new.append(offs[q] + jnp.sum(mi))
                offs = tuple(new)
            return offs

        offs = lax.fori_loop(
            0, rpt, row,
            (jnp.int32(0), jnp.int32(0), jnp.int32(0), jnp.int32(0)))

        dummy16 = dummy_base + lax.iota(jnp.int32, 16) * 8
        for q in range(4):
            off = offs[q]
            for kk in range(LANE * 8 // 16):
                pos = off + kk * 16 + lax.iota(jnp.int32, 16)
                plsc.store_scatter(
                    lbufs[q],
                    [lax.shift_right_logical(pos, 7), pos & 127], dummy16)
            ra = (off + LANE * 8 - 1) // (LANE * 8) * 8
            cb[pl.ds(0, 16)] = jnp.zeros((16,), jnp.int32) + ra
            pltpu.sync_copy(cb, couts[q].at[tid])

            def wout(cw, carry):
                pltpu.sync_copy(lbufs[q].at[pl.ds(cw * 8, 8)],
                                louts[q].at[tid, pl.ds(cw * 8, 8)])
                return carry
            lax.fori_loop(0, ra // 8, wout, 0)

    return k(pk2d)


def _sc_agg_quad(h, lsts, cnts, zeros):
    n = h.shape[0]
    w = h.shape[1]
    nh = n // 2
    za = nh + LANE
    cs = (nh // (NS * 8)) * 8
    rems = nh - NS * cs
    ca = (za // (NS * 8)) * 8
    dmask = (1 << SHIFT) - 1
    caprows = lsts[0].shape[1]
    mesh = plsc.VectorSubcoreMesh(core_axis_name="c", subcore_axis_name="s")

    @functools.partial(
        pl.kernel,
        out_type=[jax.ShapeDtypeStruct((n, w), jnp.float32),
                  jax.ShapeDtypeStruct((n, w), jnp.float32)],
        mesh=mesh,
        scratch_types=[
            pltpu.VMEM((SUP, LANE), jnp.int32),
            pltpu.VMEM((SUP, LANE), jnp.int32),
            pltpu.VMEM((SUP, LANE), jnp.int32),
            pltpu.VMEM((2, LANE, w), jnp.float32),
            pltpu.VMEM((16,), jnp.int32),
            pltpu.VMEM_SHARED((za, w), jnp.float32),
            pltpu.VMEM_SHARED((nh + 8, w), jnp.float32),
            pltpu.SemaphoreType.DMA,
            pltpu.SemaphoreType.DMA,
            pltpu.SemaphoreType.DMA,
            pltpu.SemaphoreType.DMA,
        ],
        compiler_params=pltpu.CompilerParams(needs_layout_passes=False),
    )
    def k(hr, l0, l1, l2, l3, c0r, c1r, c2r, c3r, zr, o0, o1,
          pk_sb, lsrc_sb, ldst_sb, rbuf, cntv, acc, stage,
          gs0, gs1, ss0, ss1):
        c = lax.axis_index("c")
        s = lax.axis_index("s")
        base = c * nh
        gsems = (gs0, gs1)
        ssems = (ss0, ss1)

        pltpu.sync_copy(hr.at[pl.ds(base + s * cs, cs)],
                        stage.at[pl.ds(s * cs, cs)])

        @pl.when(s == 0)
        def _():
            pltpu.sync_copy(hr.at[pl.ds(base + NS * cs, rems)],
                            stage.at[pl.ds(NS * cs, rems)])
            pltpu.sync_copy(zr.at[pl.ds(0, 8)], stage.at[pl.ds(nh, 8)])

        def do_regions(lref, cref):
            for r in range(2):
                t = s * 2 + r
                pltpu.sync_copy(cref.at[t], cntv)
                nsup_t = jnp.max(cntv[...]) // 8

                def body(sup, carry):
                    pltpu.sync_copy(lref.at[t, pl.ds(sup * SUP, SUP)], pk_sb)
                    for j in range(SUP):
                        for l in range(LANE // 16):
                            pv = pk_sb[j, pl.ds(l * 16, 16)]
                            lsrc_sb[j, pl.ds(l * 16, 16)] = (
                                lax.shift_right_logical(pv, SHIFT))
                            ldst_sb[j, pl.ds(l * 16, 16)] = pv & dmask

                    def gather(j):
                        d = pltpu.make_async_copy(stage.at[lsrc_sb.at[j]],
                                                  rbuf.at[j % 2],
                                                  gsems[j % 2])
                        d.start()
                        return d

                    gd = {0: gather(0)}
                    sd = {}
                    for j in range(SUP):
                        gd[j].wait()
                        if j >= 1:
                            sd[j - 1].wait()
                        if j + 1 < SUP:
                            gd[j + 1] = gather(j + 1)
                        d = pltpu.make_async_copy(rbuf.at[j % 2],
                                                  acc.at[ldst_sb.at[j]],
                                                  ssems[j % 2])
                        d.start(add=True)
                        sd[j] = d
                    sd[SUP - 1].wait()
                    return carry
                lax.fori_loop(0, nsup_t, body, 0)

        for p in range(2):
            pltpu.sync_copy(zr.at[pl.ds(s * ca, ca)],
                            acc.at[pl.ds(s * ca, ca)])

            @pl.when(s == 0)
            def _():
                pltpu.sync_copy(zr.at[pl.ds(NS * ca, za - NS * ca)],
                                acc.at[pl.ds(NS * ca, za - NS * ca)])
            plsc.subcore_barrier()

            @pl.when(c == 0)
            def _():
                do_regions((l0, l1)[p], (c0r, c1r)[p])

            @pl.when(c == 1)
            def _():
                do_regions((l3, l2)[p], (c3r, c2r)[p])
            plsc.subcore_barrier()

            dbase = (c if p == 0 else 1 - c) * nh

            @pl.when(c == 0)
            def _():
                pltpu.sync_copy(acc.at[pl.ds(s * cs, cs)],
                                o0.at[pl.ds(dbase + s * cs, cs)])

                @pl.when(s == 0)
                def _():
                    pltpu.sync_copy(acc.at[pl.ds(NS * cs, rems)],
                                    o0.at[pl.ds(dbase + NS * cs, rems)])

            @pl.when(c == 1)
            def _():
                pltpu.sync_copy(acc.at[pl.ds(s * cs, cs)],
                                o1.at[pl.ds(dbase + s * cs, cs)])

                @pl.when(s == 0)
                def _():
                    pltpu.sync_copy(acc.at[pl.ds(NS * cs, rems)],
                                    o1.at[pl.ds(dbase + NS * cs, rems)])
            plsc.subcore_barrier()

    return k(h, *lsts, *cnts, zeros)


def _tc_layer1(a1, x, w_rel, w_root, b):
    n = x.shape[0]
    h = w_rel.shape[1]
    bn = 1000

    def body(a1r, xr, wr, wt, br, out):
        hv = a1r[...] * wr[...] + xr[...] * wt[...] + br[...]
        out[...] = jnp.maximum(hv, 0.0)

    return pl.pallas_call(
        body,
        grid=(n // bn,),
        in_specs=[
            pl.BlockSpec((bn, 1), lambda i: (i, 0)),
            pl.BlockSpec((bn, 1), lambda i: (i, 0)),
            pl.BlockSpec((1, h), lambda i: (0, 0)),
            pl.BlockSpec((1, h), lambda i: (0, 0)),
            pl.BlockSpec((1, h), lambda i: (0, 0)),
        ],
        out_specs=pl.BlockSpec((bn, h), lambda i: (i, 0)),
        out_shape=jax.ShapeDtypeStruct((n, h), jnp.float32),
    )(a1, x, w_rel, w_root, b)


def _tc_dense(g0, g1, hin, w_rel, w_root, b, relu):
    n = g0.shape[0]
    h = w_rel.shape[0]
    bn = 1000

    def body(g0r, g1r, hr, wrr, wtr, br, out):
        g = g0r[...] + g1r[...]
        hv = (jnp.dot(g, wrr[...], preferred_element_type=jnp.float32)
              + jnp.dot(hr[...], wtr[...], preferred_element_type=jnp.float32)
              + br[...])
        if relu:
            hv = jnp.maximum(hv, 0.0)
        out[...] = hv

    return pl.pallas_call(
        body,
        grid=(n // bn,),
        in_specs=[
            pl.BlockSpec((bn, h), lambda i: (i, 0)),
            pl.BlockSpec((bn, h), lambda i: (i, 0)),
            pl.BlockSpec((bn, h), lambda i: (i, 0)),
            pl.BlockSpec((h, h), lambda i: (0, 0)),
            pl.BlockSpec((h, h), lambda i: (0, 0)),
            pl.BlockSpec((1, h), lambda i: (0, 0)),
        ],
        out_specs=pl.BlockSpec((bn, h), lambda i: (i, 0)),
        out_shape=jax.ShapeDtypeStruct((n, h), jnp.float32),
    )(g0, g1, hin, w_rel, w_root, b)


def _tc_final(g0, g1, hin, w_rel, w_root, b, batch3d, w_lin, b_lin):
    n = g0.shape[0]
    h = w_rel.shape[0]
    o = w_lin.shape[1]
    bn = 1000
    nb = n // bn

    def body(g0r, g1r, hr, wrr, wtr, br, batr, wlr, blr, out, psum, cnt):
        i = pl.program_id(0)
        g = g0r[...] + g1r[...]
        hv = (jnp.dot(g, wrr[...], preferred_element_type=jnp.float32)
              + jnp.dot(hr[...], wtr[...], preferred_element_type=jnp.float32)
              + br[...])
        ids = batr[0]
        gi = lax.broadcasted_iota(jnp.int32, (G, bn), 0)
        onehot = jnp.where(jnp.broadcast_to(ids, (G, bn)) == gi,
                           jnp.float32(1.0), jnp.float32(0.0))
        ps = jnp.dot(onehot, hv, preferred_element_type=jnp.float32)
        ct = jnp.dot(onehot, jnp.ones((bn, h), jnp.float32),
                     preferred_element_type=jnp.float32)

        @pl.when(i == 0)
        def _():
            psum[...] = ps
            cnt[...] = ct

        @pl.when(i > 0)
        def _():
            psum[...] += ps
            cnt[...] += ct

        @pl.when(i == nb - 1)
        def _():
            pooled = psum[...] / jnp.maximum(cnt[...], 1.0)
            out[...] = jnp.dot(pooled, wlr[...],
                               preferred_element_type=jnp.float32) + blr[...]

    return pl.pallas_call(
        body,
        grid=(nb,),
        in_specs=[
            pl.BlockSpec((bn, h), lambda i: (i, 0)),
            pl.BlockSpec((bn, h), lambda i: (i, 0)),
            pl.BlockSpec((bn, h), lambda i: (i, 0)),
            pl.BlockSpec((h, h), lambda i: (0, 0)),
            pl.BlockSpec((h, h), lambda i: (0, 0)),
            pl.BlockSpec((1, h), lambda i: (0, 0)),
            pl.BlockSpec((1, 1, bn), lambda i: (i, 0, 0)),
            pl.BlockSpec((h, o), lambda i: (0, 0)),
            pl.BlockSpec((1, o), lambda i: (0, 0)),
        ],
        out_specs=pl.BlockSpec((G, o), lambda i: (0, 0)),
        out_shape=jax.ShapeDtypeStruct((G, o), jnp.float32),
        scratch_shapes=[
            pltpu.VMEM((G, h), jnp.float32),
            pltpu.VMEM((G, h), jnp.float32),
        ],
    )(g0, g1, hin, w_rel, w_root, b, batch3d, w_lin, b_lin)


def kernel(x, edge_index, batch, W_rel1, b_rel1, W_root1, W_rel2, b_rel2,
           W_root2, W_rel3, b_rel3, W_root3, W_lin, b_lin):
    n = x.shape[0]
    e = edge_index.shape[1]
    h = W_rel2.shape[0]
    o = W_lin.shape[1]

    tile = NC * NS * SUP * LANE
    ep = ((e + tile - 1) // tile) * tile
    src = edge_index[0]
    dst = edge_index[1]
    srcp = jnp.concatenate(
        [src, jnp.zeros((ep - e,), jnp.int32)]).reshape(ep // LANE, LANE)
    dstp = jnp.concatenate(
        [dst, jnp.full((ep - e,), n, jnp.int32)]).reshape(ep // LANE, LANE)
    pk2d = (srcp << SHIFT) | dstp
    zeros = jnp.zeros((n + 8, h), jnp.float32)

    m = -((n + 1) // -LANE)
    m = -(m // -16) * 16
    x2d = jnp.concatenate(
        [x[:, 0], jnp.zeros((m * LANE - n,), jnp.float32)]).reshape(m, LANE)
    zeros2d = jnp.zeros((m, LANE), jnp.float32)

    b1 = b_rel1.reshape(1, h)
    b2 = b_rel2.reshape(1, h)
    b3 = b_rel3.reshape(1, h)
    bl = b_lin.reshape(1, o)
    batch3d = batch.reshape(n // 1000, 1, 1000)

    route = _sc_route(pk2d, n)
    lsts, cnts = tuple(route[:4]), tuple(route[4:])
    p0, p1 = _sc_agg_scalar(x2d, srcp, dstp, zeros2d)
    a1 = (p0 + p1).reshape(m * LANE)[:n].reshape(n, 1)
    h1 = _tc_layer1(a1, x, W_rel1, W_root1, b1)
    g20, g21 = _sc_agg_quad(h1, lsts, cnts, zeros)
    h2 = _tc_dense(g20, g21, h1, W_rel2, W_root2, b2, relu=True)
    g30, g31 = _sc_agg_quad(h2, lsts, cnts, zeros)
    return _tc_final(g30, g31, h2, W_rel3, W_root3, b3, batch3d, W_lin, bl)


# speedup vs baseline: 2.3134x; 1.0370x over previous
import functools

import jax
import jax.numpy as jnp
from jax import lax
from jax.experimental import pallas as pl
from jax.experimental.pallas import tpu as pltpu
from jax.experimental.pallas import tpu_sc as plsc

NC, NS = 2, 16
LANE = 128
SUP = 8
G = 128


def _sc_agg_scalar(x2d, src2d, dst2d, zeros2d):
    m = x2d.shape[0]
    rows = src2d.shape[0]
    rows_per_tile = rows // (NC * NS)
    nsup = rows_per_tile // SUP
    mesh = plsc.VectorSubcoreMesh(core_axis_name="c", subcore_axis_name="s")

    @functools.partial(
        pl.kernel,
        out_type=[jax.ShapeDtypeStruct((m, LANE), jnp.float32),
                  jax.ShapeDtypeStruct((m, LANE), jnp.float32)],
        mesh=mesh,
        scratch_types=[
            pltpu.VMEM((m, LANE), jnp.float32),
            pltpu.VMEM((m, LANE), jnp.float32),
            pltpu.VMEM((SUP, LANE), jnp.int32),
            pltpu.VMEM((SUP, LANE), jnp.int32),
            pltpu.VMEM((m,), jnp.int32),
            pltpu.VMEM_SHARED((m, LANE), jnp.float32),
            pltpu.SemaphoreType.DMA,
        ],
        compiler_params=pltpu.CompilerParams(needs_layout_passes=False),
    )
    def k(xr, srcr, dstr, zr, o0, o1, xloc, acc, src_sb, dst_sb, ridx, sacc,
          sem):
        c = lax.axis_index("c")
        s = lax.axis_index("s")
        tid = c * NS + s

        pltpu.sync_copy(xr, xloc)

        def zero_row(r, carry):
            for l in range(LANE // 16):
                acc[r, pl.ds(l * 16, 16)] = jnp.zeros((16,), jnp.float32)
            return carry
        lax.fori_loop(0, m, zero_row, 0)
        for kk in range(m // 16):
            ridx[pl.ds(kk * 16, 16)] = lax.iota(jnp.int32, 16) + kk * 16

        @pl.when(s == 0)
        def _():
            pltpu.sync_copy(zr, sacc)
        plsc.subcore_barrier()

        def body(sup, carry):
            r0 = tid * rows_per_tile + sup * SUP
            pltpu.sync_copy(srcr.at[pl.ds(r0, SUP)], src_sb)
            pltpu.sync_copy(dstr.at[pl.ds(r0, SUP)], dst_sb)
            for j in range(SUP):
                for l in range(LANE // 16):
                    sv = src_sb[j, pl.ds(l * 16, 16)]
                    dv = dst_sb[j, pl.ds(l * 16, 16)]
                    vals = plsc.load_gather(
                        xloc, [lax.shift_right_logical(sv, 7), sv & 127])
                    plsc.addupdate_scatter(
                        acc, [lax.shift_right_logical(dv, 7), dv & 127], vals)
            return carry
        lax.fori_loop(0, nsup, body, 0)

        pltpu.sync_copy(acc, sacc.at[ridx], add=True)
        plsc.subcore_barrier()

        @pl.when((c == 0) & (s == 0))
        def _():
            pltpu.sync_copy(sacc, o0)

        @pl.when((c == 1) & (s == 0))
        def _():
            pltpu.sync_copy(sacc, o1)

    return k(x2d, src2d, dst2d, zeros2d)


SHIFT = 14


def _sc_route(pk2d, n):
    nh = n // 2
    rows = pk2d.shape[0]
    rpt = rows // (NC * NS)
    caprows = rpt + 8
    dmask = (1 << SHIFT) - 1
    adj = [0, nh, nh << SHIFT, (nh << SHIFT) + nh]
    dummy_base = (nh << SHIFT) + nh
    mesh = plsc.VectorSubcoreMesh(core_axis_name="c", subcore_axis_name="s")

    @functools.partial(
        pl.kernel,
        out_type=(
            [jax.ShapeDtypeStruct((NC * NS, caprows, LANE), jnp.int32)] * 4
            + [jax.ShapeDtypeStruct((NC * NS, 16), jnp.int32)] * 4),
        mesh=mesh,
        scratch_types=[
            pltpu.VMEM((rpt, LANE), jnp.int32),
            pltpu.VMEM((caprows, LANE), jnp.int32),
            pltpu.VMEM((caprows, LANE), jnp.int32),
            pltpu.VMEM((caprows, LANE), jnp.int32),
            pltpu.VMEM((caprows, LANE), jnp.int32),
            pltpu.VMEM((16,), jnp.int32),
            pltpu.SemaphoreType.DMA,
        ],
        compiler_params=pltpu.CompilerParams(needs_layout_passes=False),
    )
    def k(pkr, l0, l1, l2, l3, c0, c1, c2, c3, pkb, b0, b1, b2, b3, cb, sem):
        c = lax.axis_index("c")
        s = lax.axis_index("s")
        tid = c * NS + s
        lbufs = (b0, b1, b2, b3)
        louts = (l0, l1, l2, l3)
        couts = (c0, c1, c2, c3)

        pltpu.sync_copy(pkr.at[pl.ds(tid * rpt, rpt)], pkb)

        def row(j, offs):
            for l in range(LANE // 16):
                pv = pkb[j, pl.ds(l * 16, 16)]
                sv = lax.shift_right_logical(pv, SHIFT)
                dv = pv & dmask
                ms = sv < nh
                md = dv < nh
                new = []
                for q in range(4):
                    mq = (ms if q < 2 else jnp.logical_not(ms)) & (
                        md if q % 2 == 0 else jnp.logical_not(md))
                    mi = mq.astype(jnp.int32)
                    pos = offs[q] + plsc.cumsum(mi) - 1
                    plsc.store_scatter(
                        lbufs[q],
                        [lax.shift_right_logical(pos, 7), pos & 127],
                        pv - adj[q], mask=mq)
                    new.append(offs[q] + jnp.sum(mi))
                offs = tuple(new)
            return offs

        offs = lax.fori_loop(
            0, rpt, row,
            (jnp.int32(0), jnp.int32(0), jnp.int32(0), jnp.int32(0)))

        dummy16 = dummy_base + (lax.iota(jnp.int32, 16) & 7)
        for q in range(4):
            off = offs[q]
            for kk in range(LANE * 8 // 16):
                pos = off + kk * 16 + lax.iota(jnp.int32, 16)
                plsc.store_scatter(
                    lbufs[q],
                    [lax.shift_right_logical(pos, 7), pos & 127], dummy16)
            ra = (off + LANE * 8 - 1) // (LANE * 8) * 8
            cb[pl.ds(0, 16)] = jnp.zeros((16,), jnp.int32) + ra
            pltpu.sync_copy(cb, couts[q].at[tid])

            def wout(cw, carry):
                pltpu.sync_copy(lbufs[q].at[pl.ds(cw * 8, 8)],
                                louts[q].at[tid, pl.ds(cw * 8, 8)])
                return carry
            lax.fori_loop(0, ra // 8, wout, 0)

    return k(pk2d)


def _sc_agg_quad(h, lsts, cnts, zeros):
    n = h.shape[0]
    w = h.shape[1]
    nh = n // 2
    za = nh + 8
    supa = 4
    cs = (nh // (NS * 8)) * 8
    rems = nh - NS * cs
    ca = (za // (NS * 8)) * 8
    dmask = (1 << SHIFT) - 1
    caprows = lsts[0].shape[1]
    mesh = plsc.VectorSubcoreMesh(core_axis_name="c", subcore_axis_name="s")

    @functools.partial(
        pl.kernel,
        out_type=[jax.ShapeDtypeStruct((n, w), jnp.float32),
                  jax.ShapeDtypeStruct((n, w), jnp.float32)],
        mesh=mesh,
        scratch_types=[
            pltpu.VMEM((supa, LANE), jnp.int32),
            pltpu.VMEM((supa, LANE), jnp.int32),
            pltpu.VMEM((3, LANE, w), jnp.float32),
            pltpu.VMEM((16,), jnp.int32),
            pltpu.VMEM_SHARED((za, w), jnp.float32),
            pltpu.VMEM_SHARED((nh + 8, w), jnp.float32),
            pltpu.SemaphoreType.DMA,
            pltpu.SemaphoreType.DMA,
            pltpu.SemaphoreType.DMA,
            pltpu.SemaphoreType.DMA,
        ],
        compiler_params=pltpu.CompilerParams(needs_layout_passes=False),
    )
    def k(hr, l0, l1, l2, l3, c0r, c1r, c2r, c3r, zr, o0, o1,
          pk_sb, lsrc_sb, rbuf, cntv, acc, stage,
          gs0, gs1, ss0, ss1):
        c = lax.axis_index("c")
        s = lax.axis_index("s")
        base = c * nh
        gsems = (gs0, gs1)
        ssems = (ss0, ss1)

        pltpu.sync_copy(hr.at[pl.ds(base + s * cs, cs)],
                        stage.at[pl.ds(s * cs, cs)])

        @pl.when(s == 0)
        def _():
            pltpu.sync_copy(hr.at[pl.ds(base + NS * cs, rems)],
                            stage.at[pl.ds(NS * cs, rems)])
            pltpu.sync_copy(zr.at[pl.ds(0, 8)], stage.at[pl.ds(nh, 8)])

        def do_regions(lref, cref):
            for r in range(2):
                t = s * 2 + r
                pltpu.sync_copy(cref.at[t], cntv)
                nsup_t = jnp.max(cntv[...]) // supa

                def body(sup, carry):
                    pltpu.sync_copy(lref.at[t, pl.ds(sup * supa, supa)],
                                    pk_sb)
                    for j in range(supa):
                        for l in range(LANE // 16):
                            pv = pk_sb[j, pl.ds(l * 16, 16)]
                            lsrc_sb[j, pl.ds(l * 16, 16)] = (
                                lax.shift_right_logical(pv, SHIFT))
                            pk_sb[j, pl.ds(l * 16, 16)] = pv & dmask

                    def gather(j):
                        d = pltpu.make_async_copy(stage.at[lsrc_sb.at[j]],
                                                  rbuf.at[j % 3],
                                                  gsems[j % 2])
                        d.start()
                        return d

                    gd = {0: gather(0), 1: gather(1)}
                    sd = {}
                    for j in range(supa):
                        gd[j].wait()
                        if j >= 1:
                            sd[j - 1].wait()
                        d = pltpu.make_async_copy(rbuf.at[j % 3],
                                                  acc.at[pk_sb.at[j]],
                                                  ssems[j % 2])
                        d.start(add=True)
                        sd[j] = d
                        if j + 2 < supa:
                            gd[j + 2] = gather(j + 2)
                    sd[supa - 1].wait()
                    return carry
                lax.fori_loop(0, nsup_t, body, 0)

        for p in range(2):
            pltpu.sync_copy(zr.at[pl.ds(s * ca, ca)],
                            acc.at[pl.ds(s * ca, ca)])

            @pl.when(s == 0)
            def _():
                pltpu.sync_copy(zr.at[pl.ds(NS * ca, za - NS * ca)],
                                acc.at[pl.ds(NS * ca, za - NS * ca)])
            plsc.subcore_barrier()

            @pl.when(c == 0)
            def _():
                do_regions((l0, l1)[p], (c0r, c1r)[p])

            @pl.when(c == 1)
            def _():
                do_regions((l3, l2)[p], (c3r, c2r)[p])
            plsc.subcore_barrier()

            dbase = (c if p == 0 else 1 - c) * nh

            @pl.when(c == 0)
            def _():
                pltpu.sync_copy(acc.at[pl.ds(s * cs, cs)],
                                o0.at[pl.ds(dbase + s * cs, cs)])

                @pl.when(s == 0)
                def _():
                    pltpu.sync_copy(acc.at[pl.ds(NS * cs, rems)],
                                    o0.at[pl.ds(dbase + NS * cs, rems)])

            @pl.when(c == 1)
            def _():
                pltpu.sync_copy(acc.at[pl.ds(s * cs, cs)],
                                o1.at[pl.ds(dbase + s * cs, cs)])

                @pl.when(s == 0)
                def _():
                    pltpu.sync_copy(acc.at[pl.ds(NS * cs, rems)],
                                    o1.at[pl.ds(dbase + NS * cs, rems)])
            plsc.subcore_barrier()

    return k(h, *lsts, *cnts, zeros)


def _tc_layer1(a1, x, w_rel, w_root, b):
    n = x.shape[0]
    h = w_rel.shape[1]
    bn = 1000

    def body(a1r, xr, wr, wt, br, out):
        hv = a1r[...] * wr[...] + xr[...] * wt[...] + br[...]
        out[...] = jnp.maximum(hv, 0.0)

    return pl.pallas_call(
        body,
        grid=(n // bn,),
        in_specs=[
            pl.BlockSpec((bn, 1), lambda i: (i, 0)),
            pl.BlockSpec((bn, 1), lambda i: (i, 0)),
            pl.BlockSpec((1, h), lambda i: (0, 0)),
            pl.BlockSpec((1, h), lambda i: (0, 0)),
            pl.BlockSpec((1, h), lambda i: (0, 0)),
        ],
        out_specs=pl.BlockSpec((bn, h), lambda i: (i, 0)),
        out_shape=jax.ShapeDtypeStruct((n, h), jnp.float32),
    )(a1, x, w_rel, w_root, b)


def _tc_dense(g0, g1, hin, w_rel, w_root, b, relu):
    n = g0.shape[0]
    h = w_rel.shape[0]
    bn = 1000

    def body(g0r, g1r, hr, wrr, wtr, br, out):
        g = g0r[...] + g1r[...]
        hv = (jnp.dot(g, wrr[...], preferred_element_type=jnp.float32)
              + jnp.dot(hr[...], wtr[...], preferred_element_type=jnp.float32)
              + br[...])
        if relu:
            hv = jnp.maximum(hv, 0.0)
        out[...] = hv

    return pl.pallas_call(
        body,
        grid=(n // bn,),
        in_specs=[
            pl.BlockSpec((bn, h), lambda i: (i, 0)),
            pl.BlockSpec((bn, h), lambda i: (i, 0)),
            pl.BlockSpec((bn, h), lambda i: (i, 0)),
            pl.BlockSpec((h, h), lambda i: (0, 0)),
            pl.BlockSpec((h, h), lambda i: (0, 0)),
            pl.BlockSpec((1, h), lambda i: (0, 0)),
        ],
        out_specs=pl.BlockSpec((bn, h), lambda i: (i, 0)),
        out_shape=jax.ShapeDtypeStruct((n, h), jnp.float32),
    )(g0, g1, hin, w_rel, w_root, b)


def _tc_final(g0, g1, hin, w_rel, w_root, b, batch3d, w_lin, b_lin):
    n = g0.shape[0]
    h = w_rel.shape[0]
    o = w_lin.shape[1]
    bn = 1000
    nb = n // bn

    def body(g0r, g1r, hr, wrr, wtr, br, batr, wlr, blr, out, psum, cnt):
        i = pl.program_id(0)
        g = g0r[...] + g1r[...]
        hv = (jnp.dot(g, wrr[...], preferred_element_type=jnp.float32)
              + jnp.dot(hr[...], wtr[...], preferred_element_type=jnp.float32)
              + br[...])
        ids = batr[0]
        gi = lax.broadcasted_iota(jnp.int32, (G, bn), 0)
        onehot = jnp.where(jnp.broadcast_to(ids, (G, bn)) == gi,
                           jnp.float32(1.0), jnp.float32(0.0))
        ps = jnp.dot(onehot, hv, preferred_element_type=jnp.float32)
        ct = jnp.dot(onehot, jnp.ones((bn, h), jnp.float32),
                     preferred_element_type=jnp.float32)

        @pl.when(i == 0)
        def _():
            psum[...] = ps
            cnt[...] = ct

        @pl.when(i > 0)
        def _():
            psum[...] += ps
            cnt[...] += ct

        @pl.when(i == nb - 1)
        def _():
            pooled = psum[...] / jnp.maximum(cnt[...], 1.0)
            out[...] = jnp.dot(pooled, wlr[...],
                               preferred_element_type=jnp.float32) + blr[...]

    return pl.pallas_call(
        body,
        grid=(nb,),
        in_specs=[
            pl.BlockSpec((bn, h), lambda i: (i, 0)),
            pl.BlockSpec((bn, h), lambda i: (i, 0)),
            pl.BlockSpec((bn, h), lambda i: (i, 0)),
            pl.BlockSpec((h, h), lambda i: (0, 0)),
            pl.BlockSpec((h, h), lambda i: (0, 0)),
            pl.BlockSpec((1, h), lambda i: (0, 0)),
            pl.BlockSpec((1, 1, bn), lambda i: (i, 0, 0)),
            pl.BlockSpec((h, o), lambda i: (0, 0)),
            pl.BlockSpec((1, o), lambda i: (0, 0)),
        ],
        out_specs=pl.BlockSpec((G, o), lambda i: (0, 0)),
        out_shape=jax.ShapeDtypeStruct((G, o), jnp.float32),
        scratch_shapes=[
            pltpu.VMEM((G, h), jnp.float32),
            pltpu.VMEM((G, h), jnp.float32),
        ],
    )(g0, g1, hin, w_rel, w_root, b, batch3d, w_lin, b_lin)


def kernel(x, edge_index, batch, W_rel1, b_rel1, W_root1, W_rel2, b_rel2,
           W_root2, W_rel3, b_rel3, W_root3, W_lin, b_lin):
    n = x.shape[0]
    e = edge_index.shape[1]
    h = W_rel2.shape[0]
    o = W_lin.shape[1]

    tile = NC * NS * SUP * LANE
    ep = ((e + tile - 1) // tile) * tile
    src = edge_index[0]
    dst = edge_index[1]
    srcp = jnp.concatenate(
        [src, jnp.zeros((ep - e,), jnp.int32)]).reshape(ep // LANE, LANE)
    dstp = jnp.concatenate(
        [dst, jnp.full((ep - e,), n, jnp.int32)]).reshape(ep // LANE, LANE)
    pk2d = (srcp << SHIFT) | dstp
    zeros = jnp.zeros((n + 8, h), jnp.float32)

    m = -((n + 1) // -LANE)
    m = -(m // -16) * 16
    x2d = jnp.concatenate(
        [x[:, 0], jnp.zeros((m * LANE - n,), jnp.float32)]).reshape(m, LANE)
    zeros2d = jnp.zeros((m, LANE), jnp.float32)

    b1 = b_rel1.reshape(1, h)
    b2 = b_rel2.reshape(1, h)
    b3 = b_rel3.reshape(1, h)
    bl = b_lin.reshape(1, o)
    batch3d = batch.reshape(n // 1000, 1, 1000)

    route = _sc_route(pk2d, n)
    lsts, cnts = tuple(route[:4]), tuple(route[4:])
    p0, p1 = _sc_agg_scalar(x2d, srcp, dstp, zeros2d)
    a1 = (p0 + p1).reshape(m * LANE)[:n].reshape(n, 1)
    h1 = _tc_layer1(a1, x, W_rel1, W_root1, b1)
    g20, g21 = _sc_agg_quad(h1, lsts, cnts, zeros)
    h2 = _tc_dense(g20, g21, h1, W_rel2, W_root2, b2, relu=True)
    g30, g31 = _sc_agg_quad(h2, lsts, cnts, zeros)
    return _tc_final(g30, g31, h2, W_rel3, W_root3, b3, batch3d, W_lin, bl)
